# baseline (speedup 1.0000x reference)
import dataclasses
import functools

import jax
import jax.numpy as jnp
from jax import lax
from jax.experimental import pallas as pl
from jax.experimental.pallas import tpu as pltpu
from jax.experimental.pallas import tpu_sc as plsc

_BM = 1024
_VL = 16


def _cpn_body(x_ref, kw_ref, gw_ref, gb_ref, win_ref, tab_ref,
              xn2_s, wsq_s):
    K, D = kw_ref.shape
    i = pl.program_id(0)

    @pl.when(i == 0)
    def _():
        xb = x_ref[...]
        nrm = jnp.sqrt(jnp.sum(xb * xb, axis=1, keepdims=True))
        xn = xb / jnp.maximum(nrm, 1e-12)
        xn2_s[...] = xn + xn
        kw = kw_ref[...]
        wsq_s[...] = jnp.sum(kw * kw, axis=1, keepdims=True)
        tab_ref[...] = jax.nn.sigmoid(gw_ref[...] + gb_ref[0, 0])

    xn2 = xn2_s[pl.ds(i * _BM, _BM), :]
    s2 = lax.dot_general(
        kw_ref[...], xn2, (((1,), (1,)), ((), ())),
        preferred_element_type=jnp.float32)
    negd = wsq_s[...] - s2
    winners = jnp.argmin(negd, axis=0).astype(jnp.int32)[None]
    win_ref[...] = winners[None]


def _sc_gather(tab, idx):
    B = idx.shape[0]
    K = tab.shape[0]
    info = plsc.get_sparse_core_info()
    nw = info.num_cores * info.num_subcores
    b_per_w = B // nw
    mesh = plsc.VectorSubcoreMesh(core_axis_name="c", subcore_axis_name="s")
    cp = pltpu.CompilerParams()
    if "needs_layout_passes" in pltpu.CompilerParams.__dataclass_fields__:
        cp = dataclasses.replace(cp, needs_layout_passes=False)

    @functools.partial(
        pl.kernel, mesh=mesh, compiler_params=cp,
        out_type=jax.ShapeDtypeStruct((B,), jnp.float32),
        scratch_types=[
            pltpu.VMEM((K,), jnp.float32),
            pltpu.VMEM((b_per_w,), jnp.int32),
            pltpu.VMEM((b_per_w,), jnp.float32),
        ],
    )
    def k(tab_hbm, idx_hbm, out_hbm, tab_v, idx_v, out_v):
        wid = lax.axis_index("s") * info.num_cores + lax.axis_index("c")
        base = wid * b_per_w
        pltpu.sync_copy(tab_hbm, tab_v)
        pltpu.sync_copy(idx_hbm.at[pl.ds(base, b_per_w)], idx_v)

        @pl.loop(0, b_per_w, step=_VL)
        def _(j):
            idx16 = idx_v.at[pl.ds(j, _VL)][...]
            out_v.at[pl.ds(j, _VL)][...] = plsc.load_gather(tab_v, [idx16])

        pltpu.sync_copy(out_v, out_hbm.at[pl.ds(base, b_per_w)])

    return k(tab, idx)


def kernel(x, kohonen_weights, grossberg_w, grossberg_b):
    B, D = x.shape
    K = kohonen_weights.shape[0]
    G = B // _BM
    gw_col = grossberg_w.reshape(K, 1)
    gb = grossberg_b.reshape(1, 1)
    win, tab = pl.pallas_call(
        _cpn_body,
        grid=(G,),
        in_specs=[
            pl.BlockSpec((B, D), lambda i: (0, 0)),
            pl.BlockSpec((K, D), lambda i: (0, 0)),
            pl.BlockSpec((K, 1), lambda i: (0, 0)),
            pl.BlockSpec((1, 1), lambda i: (0, 0)),
        ],
        scratch_shapes=[
            pltpu.VMEM((B, D), jnp.float32),
            pltpu.VMEM((K, 1), jnp.float32),
        ],
        out_specs=[
            pl.BlockSpec((1, 1, _BM), lambda i: (i, 0, 0)),
            pl.BlockSpec((K, 1), lambda i: (0, 0)),
        ],
        out_shape=[
            jax.ShapeDtypeStruct((G, 1, _BM), jnp.int32),
            jax.ShapeDtypeStruct((K, 1), jnp.float32),
        ],
    )(x, kohonen_weights, gw_col, gb)
    winners = win.reshape(B)
    out = _sc_gather(tab.reshape(K), winners)
    return out.reshape(B, 1), winners


# speedup vs baseline: 1.3109x; 1.3109x over previous
import dataclasses
import functools

import jax
import jax.numpy as jnp
from jax import lax
from jax.experimental import pallas as pl
from jax.experimental.pallas import tpu as pltpu
from jax.experimental.pallas import tpu_sc as plsc

_BM = 1024
_VL = 16


def _cpn_body(x_ref, kw_ref, gw_ref, gb_ref, win_ref, out_ref,
              xn2_s, wsq_s, tab_s):
    K, D = kw_ref.shape
    i = pl.program_id(0)

    @pl.when(i == 0)
    def _():
        xb = x_ref[...]
        nrm = jnp.sqrt(jnp.sum(xb * xb, axis=1, keepdims=True))
        xn = xb / jnp.maximum(nrm, 1e-12)
        xn2_s[...] = xn + xn
        kw = kw_ref[...]
        wsq_s[...] = jnp.sum(kw * kw, axis=1, keepdims=True)
        tab_s[...] = jax.nn.sigmoid(gw_ref[...] + gb_ref[0, 0])

    xn2 = xn2_s[pl.ds(i * _BM, _BM), :]
    s2 = lax.dot_general(
        kw_ref[...], xn2, (((1,), (1,)), ((), ())),
        preferred_element_type=jnp.float32)
    negd = wsq_s[...] - s2
    winners = jnp.argmin(negd, axis=0).astype(jnp.int32)[None]
    win_ref[...] = winners[None]
    rows = lax.broadcasted_iota(jnp.int32, (K, _BM), 0)
    onehot = jnp.where(rows == winners, 1.0, 0.0)
    outcol = lax.dot_general(
        tab_s[...], onehot, (((0,), (0,)), ((), ())),
        preferred_element_type=jnp.float32)
    out_ref[...] = outcol[None]


def _sc_gather(tab, idx):
    B = idx.shape[0]
    K = tab.shape[0]
    info = plsc.get_sparse_core_info()
    nw = info.num_cores * info.num_subcores
    b_per_w = B // nw
    mesh = plsc.VectorSubcoreMesh(core_axis_name="c", subcore_axis_name="s")
    cp = pltpu.CompilerParams()
    if "needs_layout_passes" in pltpu.CompilerParams.__dataclass_fields__:
        cp = dataclasses.replace(cp, needs_layout_passes=False)

    @functools.partial(
        pl.kernel, mesh=mesh, compiler_params=cp,
        out_type=jax.ShapeDtypeStruct((B,), jnp.float32),
        scratch_types=[
            pltpu.VMEM((K,), jnp.float32),
            pltpu.VMEM((b_per_w,), jnp.int32),
            pltpu.VMEM((b_per_w,), jnp.float32),
        ],
    )
    def k(tab_hbm, idx_hbm, out_hbm, tab_v, idx_v, out_v):
        wid = lax.axis_index("s") * info.num_cores + lax.axis_index("c")
        base = wid * b_per_w
        pltpu.sync_copy(tab_hbm, tab_v)
        pltpu.sync_copy(idx_hbm.at[pl.ds(base, b_per_w)], idx_v)

        @pl.loop(0, b_per_w, step=_VL)
        def _(j):
            idx16 = idx_v.at[pl.ds(j, _VL)][...]
            out_v.at[pl.ds(j, _VL)][...] = plsc.load_gather(tab_v, [idx16])

        pltpu.sync_copy(out_v, out_hbm.at[pl.ds(base, b_per_w)])

    return k(tab, idx)


def kernel(x, kohonen_weights, grossberg_w, grossberg_b):
    B, D = x.shape
    K = kohonen_weights.shape[0]
    G = B // _BM
    gw_col = grossberg_w.reshape(K, 1)
    gb = grossberg_b.reshape(1, 1)
    win, out = pl.pallas_call(
        _cpn_body,
        grid=(G,),
        in_specs=[
            pl.BlockSpec((B, D), lambda i: (0, 0)),
            pl.BlockSpec((K, D), lambda i: (0, 0)),
            pl.BlockSpec((K, 1), lambda i: (0, 0)),
            pl.BlockSpec((1, 1), lambda i: (0, 0)),
        ],
        scratch_shapes=[
            pltpu.VMEM((B, D), jnp.float32),
            pltpu.VMEM((K, 1), jnp.float32),
            pltpu.VMEM((K, 1), jnp.float32),
        ],
        out_specs=[
            pl.BlockSpec((1, 1, _BM), lambda i: (i, 0, 0)),
            pl.BlockSpec((1, 1, _BM), lambda i: (i, 0, 0)),
        ],
        out_shape=[
            jax.ShapeDtypeStruct((G, 1, _BM), jnp.int32),
            jax.ShapeDtypeStruct((G, 1, _BM), jnp.float32),
        ],
    )(x, kohonen_weights, gw_col, gb)
    winners = win.reshape(B)
    return out.reshape(B, 1), winners


# speedup vs baseline: 1.3132x; 1.0018x over previous
import jax
import jax.numpy as jnp
from jax import lax
from jax.experimental import pallas as pl
from jax.experimental.pallas import tpu as pltpu

_BM = 1024


def _cpn_body(x_ref, kw_ref, gw_ref, gb_ref, win_ref, out_ref,
              xn2_s, wsq_s, tab_s):
    K, D = kw_ref.shape
    i = pl.program_id(0)

    @pl.when(i == 0)
    def _():
        xb = x_ref[...]
        nrm = jnp.sqrt(jnp.sum(xb * xb, axis=1, keepdims=True))
        xn = xb / jnp.maximum(nrm, 1e-12)
        xn2_s[...] = xn + xn
        kw = kw_ref[...]
        wsq_s[...] = jnp.sum(kw * kw, axis=1, keepdims=True)
        tab_s[...] = jax.nn.sigmoid(gw_ref[...] + gb_ref[0, 0])

    xn2 = xn2_s[pl.ds(i * _BM, _BM), :]
    s2 = lax.dot_general(
        kw_ref[...], xn2, (((1,), (1,)), ((), ())),
        preferred_element_type=jnp.float32)
    negd = wsq_s[...] - s2
    winners = jnp.argmin(negd, axis=0).astype(jnp.int32)[None]
    win_ref[...] = winners[None]
    rows = lax.broadcasted_iota(jnp.int32, (K, _BM), 0)
    onehot = jnp.where(rows == winners, 1.0, 0.0)
    outcol = lax.dot_general(
        tab_s[...], onehot, (((0,), (0,)), ((), ())),
        preferred_element_type=jnp.float32)
    out_ref[...] = outcol[None]


def kernel(x, kohonen_weights, grossberg_w, grossberg_b):
    B, D = x.shape
    K = kohonen_weights.shape[0]
    G = B // _BM
    gw_col = grossberg_w.reshape(K, 1)
    gb = grossberg_b.reshape(1, 1)
    win, out = pl.pallas_call(
        _cpn_body,
        grid=(G,),
        in_specs=[
            pl.BlockSpec((B, D), lambda i: (0, 0)),
            pl.BlockSpec((K, D), lambda i: (0, 0)),
            pl.BlockSpec((K, 1), lambda i: (0, 0)),
            pl.BlockSpec((1, 1), lambda i: (0, 0)),
        ],
        scratch_shapes=[
            pltpu.VMEM((B, D), jnp.float32),
            pltpu.VMEM((K, 1), jnp.float32),
            pltpu.VMEM((K, 1), jnp.float32),
        ],
        out_specs=[
            pl.BlockSpec((1, 1, _BM), lambda i: (i, 0, 0)),
            pl.BlockSpec((1, 1, _BM), lambda i: (i, 0, 0)),
        ],
        out_shape=[
            jax.ShapeDtypeStruct((G, 1, _BM), jnp.int32),
            jax.ShapeDtypeStruct((G, 1, _BM), jnp.float32),
        ],
    )(x, kohonen_weights, gw_col, gb)
    winners = win.reshape(B)
    return out.reshape(B, 1), winners
